# baseline (speedup 1.0000x reference)
import functools

import jax
import jax.numpy as jnp
from jax import lax
from jax.experimental import pallas as pl
from jax.experimental.pallas import tpu as pltpu
from jax.experimental.pallas import tpu_sc as plsc

_NUMS = (3, 4, 5, 6, 7, 8, 9, 14, 15, 16, 17, 18, 22, 23, 24, 25, 26, 27,
         28, 29, 30, 31, 37, 38, 39, 46, 47, 48, 49, 50, 51, 52, 53, 54,
         55, 56, 57)

def _signed32(u):
    return u - (1 << 32) if u >= (1 << 31) else u

_MASK_LO = _signed32(sum(1 << n for n in _NUMS if n < 32))
_MASK_HI = _signed32(sum(1 << (n - 32) for n in _NUMS if n >= 32))

_NC = 2
_NS = 16
_NW = _NC * _NS
_L = 16

_ROWS = 64
_COLS = 32768

_R_SC = 16
_R_TC = _ROWS - _R_SC
_ROW0_SC = _R_TC

_SEG = _R_SC * _COLS // _NW
_CH = 8192
_NCHUNK = _SEG // _CH
_NBUF = 2


def _sc_body(x_hbm, out_hbm, *scratch):
    ins = scratch[0:_NBUF]
    outs = scratch[_NBUF:2 * _NBUF]
    isems = scratch[2 * _NBUF:3 * _NBUF]
    osems = scratch[3 * _NBUF:4 * _NBUF]
    wid = lax.axis_index("s") * _NC + lax.axis_index("c")
    seg0 = wid * _SEG
    lo_vec = jnp.full((_L,), _MASK_LO, jnp.int32)
    hi_vec = jnp.full((_L,), _MASK_HI, jnp.int32)

    def _in_copy(c):
        off = seg0 + c * _CH
        return pltpu.async_copy(
            x_hbm.at[_ROW0_SC + off // _COLS, pl.ds(off % _COLS, _CH)],
            ins[c % _NBUF], isems[c % _NBUF])

    def _out_copy(c):
        off = seg0 + c * _CH
        return pltpu.async_copy(
            outs[c % _NBUF],
            out_hbm.at[off // _COLS, pl.ds(off % _COLS, _CH)],
            osems[c % _NBUF])

    h_in = {c: _in_copy(c) for c in range(min(_NBUF, _NCHUNK))}
    h_out = {}
    for c in range(_NCHUNK):
        h_in[c].wait()
        if c >= _NBUF:
            h_out[c - _NBUF].wait()
        src = ins[c % _NBUF]
        dst = outs[c % _NBUF]

        @plsc.parallel_loop(0, _CH, step=_L, unroll=8)
        def _compute(i):
            v = src[pl.ds(i, _L)]
            word = jnp.where(v < 32, lo_vec, hi_vec)
            bit = lax.shift_right_logical(word, v & 31) & 1
            dst[pl.ds(i, _L)] = lax.shift_left(v, bit)

        h_out[c] = _out_copy(c)
        if c + _NBUF < _NCHUNK:
            h_in[c + _NBUF] = _in_copy(c + _NBUF)
    for c in range(max(0, _NCHUNK - _NBUF), _NCHUNK):
        h_out[c].wait()


@functools.cache
def _sc_call():
    return functools.partial(
        pl.kernel,
        out_type=jax.ShapeDtypeStruct((_R_SC, _COLS), jnp.int32),
        compiler_params=pltpu.CompilerParams(skip_device_barrier=True),
        mesh=plsc.VectorSubcoreMesh(
            core_axis_name="c", subcore_axis_name="s",
            num_cores=_NC, num_subcores=_NS),
        scratch_types=(
            [pltpu.VMEM((_CH,), jnp.int32) for _ in range(2 * _NBUF)]
            + [pltpu.SemaphoreType.DMA for _ in range(2 * _NBUF)]
        ),
    )(_sc_body)


_BR = 24
_BC = _COLS


def _tc_body(x_ref, o_ref):
    v = x_ref[...]
    word = jnp.where(v < 32, jnp.int32(_MASK_LO), jnp.int32(_MASK_HI))
    bit = lax.shift_right_logical(word, v & 31) & 1
    o_ref[...] = lax.shift_left(v, bit)


@functools.cache
def _tc_call():
    return pl.pallas_call(
        _tc_body,
        grid=(_R_TC // _BR,),
        in_specs=[pl.BlockSpec((_BR, _BC), lambda i: (i, 0))],
        out_specs=pl.BlockSpec((_BR, _BC), lambda i: (i, 0)),
        out_shape=jax.ShapeDtypeStruct((_ROWS, _COLS), jnp.int32),
    )


@jax.jit
def kernel(x):
    sc_out = _sc_call()(x)
    tc_out = _tc_call()(x)
    return lax.dynamic_update_slice(tc_out, sc_out, (_ROW0_SC, 0))


# speedup vs baseline: 1.0089x; 1.0089x over previous
import functools

import jax
import jax.numpy as jnp
from jax import lax
from jax.experimental import pallas as pl
from jax.experimental.pallas import tpu as pltpu
from jax.experimental.pallas import tpu_sc as plsc

_NUMS = (3, 4, 5, 6, 7, 8, 9, 14, 15, 16, 17, 18, 22, 23, 24, 25, 26, 27,
         28, 29, 30, 31, 37, 38, 39, 46, 47, 48, 49, 50, 51, 52, 53, 54,
         55, 56, 57)

def _signed32(u):
    return u - (1 << 32) if u >= (1 << 31) else u

_MASK_LO = _signed32(sum(1 << n for n in _NUMS if n < 32))
_MASK_HI = _signed32(sum(1 << (n - 32) for n in _NUMS if n >= 32))

_NC = 2
_NS = 16
_NW = _NC * _NS
_L = 16

_ROWS = 64
_COLS = 32768

_R_SC = 16
_R_TC = _ROWS - _R_SC
_ROW0_SC = _R_TC

_SEG = _R_SC * _COLS // _NW
_CH = 8192
_NCHUNK = _SEG // _CH
_NBUF = 2


def _sc_body(x_hbm, out_hbm, *scratch):
    ins = scratch[0:_NBUF]
    outs = scratch[_NBUF:2 * _NBUF]
    isems = scratch[2 * _NBUF:3 * _NBUF]
    osems = scratch[3 * _NBUF:4 * _NBUF]
    wid = lax.axis_index("s") * _NC + lax.axis_index("c")
    seg0 = wid * _SEG
    lo_vec = jnp.full((_L,), _MASK_LO, jnp.int32)
    hi_vec = jnp.full((_L,), _MASK_HI, jnp.int32)

    def _in_copy(c):
        off = seg0 + c * _CH
        return pltpu.async_copy(
            x_hbm.at[_ROW0_SC + off // _COLS, pl.ds(off % _COLS, _CH)],
            ins[c % _NBUF], isems[c % _NBUF])

    def _out_copy(c):
        off = seg0 + c * _CH
        return pltpu.async_copy(
            outs[c % _NBUF],
            out_hbm.at[off // _COLS, pl.ds(off % _COLS, _CH)],
            osems[c % _NBUF])

    h_in = {c: _in_copy(c) for c in range(min(_NBUF, _NCHUNK))}
    h_out = {}
    for c in range(_NCHUNK):
        h_in[c].wait()
        if c >= _NBUF:
            h_out[c - _NBUF].wait()
        src = ins[c % _NBUF]
        dst = outs[c % _NBUF]

        @plsc.parallel_loop(0, _CH, step=_L, unroll=8)
        def _compute(i):
            v = src[pl.ds(i, _L)]
            word = jnp.where(v < 32, lo_vec, hi_vec)
            bit = lax.shift_right_logical(word, v & 31) & 1
            dst[pl.ds(i, _L)] = lax.shift_left(v, bit)

        h_out[c] = _out_copy(c)
        if c + _NBUF < _NCHUNK:
            h_in[c + _NBUF] = _in_copy(c + _NBUF)
    for c in range(max(0, _NCHUNK - _NBUF), _NCHUNK):
        h_out[c].wait()


@functools.cache
def _sc_call():
    return functools.partial(
        pl.kernel,
        out_type=jax.ShapeDtypeStruct((_R_SC, _COLS), jnp.int32),
        mesh=plsc.VectorSubcoreMesh(
            core_axis_name="c", subcore_axis_name="s",
            num_cores=_NC, num_subcores=_NS),
        scratch_types=(
            [pltpu.VMEM((_CH,), jnp.int32) for _ in range(2 * _NBUF)]
            + [pltpu.SemaphoreType.DMA for _ in range(2 * _NBUF)]
        ),
    )(_sc_body)


_BR = 8
_BC = _COLS


def _tc_body(x_ref, o_ref):
    v = x_ref[...]
    word = jnp.where(v < 32, jnp.int32(_MASK_LO), jnp.int32(_MASK_HI))
    bit = lax.shift_right_logical(word, v & 31) & 1
    o_ref[...] = lax.shift_left(v, bit)


@functools.cache
def _tc_call():
    return pl.pallas_call(
        _tc_body,
        grid=(_R_TC // _BR,),
        in_specs=[pl.BlockSpec((_BR, _BC), lambda i: (i, 0))],
        out_specs=pl.BlockSpec((_BR, _BC), lambda i: (i, 0)),
        out_shape=jax.ShapeDtypeStruct((_ROWS, _COLS), jnp.int32),
    )


@jax.jit
def kernel(x):
    sc_out = _sc_call()(x)
    tc_out = _tc_call()(x)
    return lax.dynamic_update_slice(tc_out, sc_out, (_ROW0_SC, 0))


# speedup vs baseline: 1.0105x; 1.0015x over previous
import functools

import jax
import jax.numpy as jnp
from jax import lax
from jax.experimental import pallas as pl
from jax.experimental.pallas import tpu as pltpu
from jax.experimental.pallas import tpu_sc as plsc

_NUMS = (3, 4, 5, 6, 7, 8, 9, 14, 15, 16, 17, 18, 22, 23, 24, 25, 26, 27,
         28, 29, 30, 31, 37, 38, 39, 46, 47, 48, 49, 50, 51, 52, 53, 54,
         55, 56, 57)

def _signed32(u):
    return u - (1 << 32) if u >= (1 << 31) else u

_MASK_LO = _signed32(sum(1 << n for n in _NUMS if n < 32))
_MASK_HI = _signed32(sum(1 << (n - 32) for n in _NUMS if n >= 32))

_NC = 2
_NS = 16
_NW = _NC * _NS
_L = 16

_ROWS = 64
_COLS = 32768

_R_SC = 16
_R_TC = _ROWS - _R_SC
_ROW0_SC = _R_TC

_SEG = _R_SC * _COLS // _NW
_CH = 8192
_NCHUNK = _SEG // _CH
_NBUF = 2


def _sc_body(x_hbm, out_hbm, *scratch):
    ins = scratch[0:_NBUF]
    outs = scratch[_NBUF:2 * _NBUF]
    isems = scratch[2 * _NBUF:3 * _NBUF]
    osems = scratch[3 * _NBUF:4 * _NBUF]
    wid = lax.axis_index("s") * _NC + lax.axis_index("c")
    seg0 = wid * _SEG
    lo_vec = jnp.full((_L,), _MASK_LO, jnp.int32)
    hi_vec = jnp.full((_L,), _MASK_HI, jnp.int32)

    def _in_copy(c):
        off = seg0 + c * _CH
        return pltpu.async_copy(
            x_hbm.at[_ROW0_SC + off // _COLS, pl.ds(off % _COLS, _CH)],
            ins[c % _NBUF], isems[c % _NBUF])

    def _out_copy(c):
        off = seg0 + c * _CH
        return pltpu.async_copy(
            outs[c % _NBUF],
            out_hbm.at[off // _COLS, pl.ds(off % _COLS, _CH)],
            osems[c % _NBUF])

    h_in = {c: _in_copy(c) for c in range(min(_NBUF, _NCHUNK))}
    h_out = {}
    for c in range(_NCHUNK):
        h_in[c].wait()
        if c >= _NBUF:
            h_out[c - _NBUF].wait()
        src = ins[c % _NBUF]
        dst = outs[c % _NBUF]

        @plsc.parallel_loop(0, _CH, step=_L, unroll=8)
        def _compute(i):
            v = src[pl.ds(i, _L)]
            word = jnp.where(v < 32, lo_vec, hi_vec)
            bit = lax.shift_right_logical(word, v & 31) & 1
            dst[pl.ds(i, _L)] = lax.shift_left(v, bit)

        h_out[c] = _out_copy(c)
        if c + _NBUF < _NCHUNK:
            h_in[c + _NBUF] = _in_copy(c + _NBUF)
    for c in range(max(0, _NCHUNK - _NBUF), _NCHUNK):
        h_out[c].wait()


@functools.cache
def _sc_call():
    return functools.partial(
        pl.kernel,
        out_type=jax.ShapeDtypeStruct((_R_SC, _COLS), jnp.int32),
        mesh=plsc.VectorSubcoreMesh(
            core_axis_name="c", subcore_axis_name="s",
            num_cores=_NC, num_subcores=_NS),
        scratch_types=(
            [pltpu.VMEM((_CH,), jnp.int32) for _ in range(2 * _NBUF)]
            + [pltpu.SemaphoreType.DMA for _ in range(2 * _NBUF)]
        ),
    )(_sc_body)


_BR = 8
_BC = _COLS


_NTB = _R_TC // _BR


def _tc_body(x_ref, o_ref, in0, in1, ou0, ou1, is0, is1, os0, os1):
    ins = (in0, in1)
    outs = (ou0, ou1)
    isems = (is0, is1)
    osems = (os0, os1)

    def _in_copy(b):
        return pltpu.async_copy(
            x_ref.at[pl.ds(b * _BR, _BR), :], ins[b % 2], isems[b % 2])

    def _out_copy(b):
        return pltpu.async_copy(
            outs[b % 2], o_ref.at[pl.ds(b * _BR, _BR), :], osems[b % 2])

    h_in = {b: _in_copy(b) for b in range(2)}
    h_out = {}
    for b in range(_NTB):
        h_in[b].wait()
        if b >= 2:
            h_out[b - 2].wait()
        v = ins[b % 2][...]
        word = jnp.where(v < 32, jnp.int32(_MASK_LO), jnp.int32(_MASK_HI))
        bit = lax.shift_right_logical(word, v & 31) & 1
        outs[b % 2][...] = lax.shift_left(v, bit)
        h_out[b] = _out_copy(b)
        if b + 2 < _NTB:
            h_in[b + 2] = _in_copy(b + 2)
    h_out[_NTB - 2].wait()
    h_out[_NTB - 1].wait()


@functools.cache
def _tc_call():
    return pl.pallas_call(
        _tc_body,
        in_specs=[pl.BlockSpec(memory_space=pl.ANY)],
        out_specs=pl.BlockSpec(memory_space=pl.ANY),
        out_shape=jax.ShapeDtypeStruct((_ROWS, _COLS), jnp.int32),
        scratch_shapes=(
            [pltpu.VMEM((_BR, _BC), jnp.int32) for _ in range(4)]
            + [pltpu.SemaphoreType.DMA for _ in range(4)]
        ),
    )


@jax.jit
def kernel(x):
    sc_out = _sc_call()(x)
    tc_out = _tc_call()(x)
    return lax.dynamic_update_slice(tc_out, sc_out, (_ROW0_SC, 0))


# speedup vs baseline: 1.0286x; 1.0180x over previous
import functools

import jax
import jax.numpy as jnp
from jax import lax
from jax.experimental import pallas as pl
from jax.experimental.pallas import tpu as pltpu
from jax.experimental.pallas import tpu_sc as plsc

_NUMS = (3, 4, 5, 6, 7, 8, 9, 14, 15, 16, 17, 18, 22, 23, 24, 25, 26, 27,
         28, 29, 30, 31, 37, 38, 39, 46, 47, 48, 49, 50, 51, 52, 53, 54,
         55, 56, 57)

def _signed32(u):
    return u - (1 << 32) if u >= (1 << 31) else u

_MASK_LO = _signed32(sum(1 << n for n in _NUMS if n < 32))
_MASK_HI = _signed32(sum(1 << (n - 32) for n in _NUMS if n >= 32))

_NC = 2
_NS = 16
_NW = _NC * _NS
_L = 16

_ROWS = 64
_COLS = 32768

_R_SC = 8
_R_TC = _ROWS - _R_SC
_ROW0_SC = _R_TC

_SEG = _R_SC * _COLS // _NW
_CH = 8192
_NCHUNK = _SEG // _CH
_NBUF = 2


def _sc_body(x_hbm, out_hbm, *scratch):
    ins = scratch[0:_NBUF]
    outs = scratch[_NBUF:2 * _NBUF]
    isems = scratch[2 * _NBUF:3 * _NBUF]
    osems = scratch[3 * _NBUF:4 * _NBUF]
    wid = lax.axis_index("s") * _NC + lax.axis_index("c")
    seg0 = wid * _SEG
    lo_vec = jnp.full((_L,), _MASK_LO, jnp.int32)
    hi_vec = jnp.full((_L,), _MASK_HI, jnp.int32)

    def _in_copy(c):
        off = seg0 + c * _CH
        return pltpu.async_copy(
            x_hbm.at[_ROW0_SC + off // _COLS, pl.ds(off % _COLS, _CH)],
            ins[c % _NBUF], isems[c % _NBUF])

    def _out_copy(c):
        off = seg0 + c * _CH
        return pltpu.async_copy(
            outs[c % _NBUF],
            out_hbm.at[off // _COLS, pl.ds(off % _COLS, _CH)],
            osems[c % _NBUF])

    h_in = {c: _in_copy(c) for c in range(min(_NBUF, _NCHUNK))}
    h_out = {}
    for c in range(_NCHUNK):
        h_in[c].wait()
        if c >= _NBUF:
            h_out[c - _NBUF].wait()
        src = ins[c % _NBUF]
        dst = outs[c % _NBUF]

        @plsc.parallel_loop(0, _CH, step=_L, unroll=8)
        def _compute(i):
            v = src[pl.ds(i, _L)]
            word = jnp.where(v < 32, lo_vec, hi_vec)
            bit = lax.shift_right_logical(word, v & 31) & 1
            dst[pl.ds(i, _L)] = lax.shift_left(v, bit)

        h_out[c] = _out_copy(c)
        if c + _NBUF < _NCHUNK:
            h_in[c + _NBUF] = _in_copy(c + _NBUF)
    for c in range(max(0, _NCHUNK - _NBUF), _NCHUNK):
        h_out[c].wait()


@functools.cache
def _sc_call():
    return functools.partial(
        pl.kernel,
        out_type=jax.ShapeDtypeStruct((_R_SC, _COLS), jnp.int32),
        mesh=plsc.VectorSubcoreMesh(
            core_axis_name="c", subcore_axis_name="s",
            num_cores=_NC, num_subcores=_NS),
        scratch_types=(
            [pltpu.VMEM((_CH,), jnp.int32) for _ in range(2 * _NBUF)]
            + [pltpu.SemaphoreType.DMA for _ in range(2 * _NBUF)]
        ),
    )(_sc_body)


_BR = 8
_BC = _COLS


_NTB = _R_TC // _BR


def _tc_body(x_ref, o_ref, in0, in1, ou0, ou1, is0, is1, os0, os1):
    ins = (in0, in1)
    outs = (ou0, ou1)
    isems = (is0, is1)
    osems = (os0, os1)

    def _in_copy(b):
        return pltpu.async_copy(
            x_ref.at[pl.ds(b * _BR, _BR), :], ins[b % 2], isems[b % 2])

    def _out_copy(b):
        return pltpu.async_copy(
            outs[b % 2], o_ref.at[pl.ds(b * _BR, _BR), :], osems[b % 2])

    h_in = {b: _in_copy(b) for b in range(2)}
    h_out = {}
    for b in range(_NTB):
        h_in[b].wait()
        if b >= 2:
            h_out[b - 2].wait()
        v = ins[b % 2][...]
        word = jnp.where(v < 32, jnp.int32(_MASK_LO), jnp.int32(_MASK_HI))
        bit = lax.shift_right_logical(word, v & 31) & 1
        outs[b % 2][...] = lax.shift_left(v, bit)
        h_out[b] = _out_copy(b)
        if b + 2 < _NTB:
            h_in[b + 2] = _in_copy(b + 2)
    h_out[_NTB - 2].wait()
    h_out[_NTB - 1].wait()


@functools.cache
def _tc_call():
    return pl.pallas_call(
        _tc_body,
        in_specs=[pl.BlockSpec(memory_space=pl.ANY)],
        out_specs=pl.BlockSpec(memory_space=pl.ANY),
        out_shape=jax.ShapeDtypeStruct((_ROWS, _COLS), jnp.int32),
        scratch_shapes=(
            [pltpu.VMEM((_BR, _BC), jnp.int32) for _ in range(4)]
            + [pltpu.SemaphoreType.DMA for _ in range(4)]
        ),
    )


@jax.jit
def kernel(x):
    sc_out = _sc_call()(x)
    tc_out = _tc_call()(x)
    return lax.dynamic_update_slice(tc_out, sc_out, (_ROW0_SC, 0))
